# Initial kernel scaffold; baseline (speedup 1.0000x reference)
import jax
import jax.numpy as jnp
from jax.experimental import pallas as pl


def kernel(inputs, w):
    raise NotImplementedError("write your pallas kernel here")


# speedup vs baseline: 7.7002x; 7.7002x over previous
import functools

import jax
import jax.numpy as jnp
from jax import lax
from jax.experimental import pallas as pl
from jax.experimental.pallas import tpu as pltpu
from jax.experimental.pallas import tpu_sc as plsc

B = 16384
L = 50
DIM = 32
VOCAB = 81616

NC = 2
NS = 16
NW = NC * NS

GPC = 2
CLEN = GPC * L + 4
NCHUNK = B // GPC
CPW = NCHUNK // NW
RPW = B // NW


def _sc_body(w_hbm, idx_hbm, out_hbm, idx_v, rows_v, acc_v, sems):
    wid = lax.axis_index("s") * NC + lax.axis_index("c")
    chunk0 = wid * CPW

    pltpu.sync_copy(idx_hbm.at[pl.ds(chunk0, CPW)], idx_v)

    def start(k, b):
        pltpu.async_copy(w_hbm.at[idx_v.at[k]], rows_v.at[b], sems.at[b])

    def wait(b):
        pltpu.make_async_copy(w_hbm.at[idx_v.at[0]], rows_v.at[b], sems.at[b]).wait()

    start(0, 0)
    start(1, 1)

    def pair_body(j, carry):
        for b in range(2):
            k = 2 * j + b
            wait(b)
            for g in range(GPC):
                for h in range(2):
                    v = rows_v[b, g * L, pl.ds(h * 16, 16)]
                    for r in range(1, L):
                        v = v + rows_v[b, g * L + r, pl.ds(h * 16, 16)]
                    acc_v[k * GPC + g, pl.ds(h * 16, 16)] = v

            @pl.when(j < CPW // 2 - 1)
            def _():
                start(k + 2, b)

        return carry

    lax.fori_loop(0, CPW // 2, pair_body, 0)

    pltpu.sync_copy(acc_v, out_hbm.at[pl.ds(wid * RPW, RPW)])


@jax.jit
def _sc_embed_sum(w, idx_chunks):
    mesh = plsc.VectorSubcoreMesh(core_axis_name="c", subcore_axis_name="s")
    return pl.kernel(
        _sc_body,
        out_type=jax.ShapeDtypeStruct((B, DIM), jnp.float32),
        mesh=mesh,
        scratch_types=[
            pltpu.VMEM((CPW, CLEN), jnp.int32),
            pltpu.VMEM((2, CLEN, DIM), jnp.float32),
            pltpu.VMEM((RPW, DIM), jnp.float32),
            pltpu.SemaphoreType.DMA((2,)),
        ],
        compiler_params=pltpu.CompilerParams(use_tc_tiling_on_sc=False),
    )(w, idx_chunks)


def kernel(inputs, w):
    idx = inputs.astype(jnp.int32).reshape(NCHUNK, GPC * L)
    idx_chunks = jnp.pad(idx, ((0, 0), (0, CLEN - GPC * L)))
    return _sc_embed_sum(w, idx_chunks)
